# Initial kernel scaffold; baseline (speedup 1.0000x reference)
import jax
import jax.numpy as jnp
from jax.experimental import pallas as pl


def kernel(x, edge_index, W1r, W1n, b1, W2r, W2n, b2, Wp, bp):
    raise NotImplementedError("write your pallas kernel here")


# speedup vs baseline: 9.1804x; 9.1804x over previous
import jax
import jax.numpy as jnp
from jax import lax
from jax.experimental import pallas as pl
from jax.experimental.pallas import tpu as pltpu
from jax.experimental.pallas import tpu_sc as plsc

N = 10000
E = 320000
D = 128
H = 128
A = 2

CB = 128
NCHUNK = 2560
E_PAD = NCHUNK * CB
NSUB = 16
ND = NCHUNK // NSUB
HALFC = NCHUNK // 2
NH = HALFC // NSUB
NPAD = 10240
RPT = NPAD // NSUB
R = 640
G = NPAD // R


def _sc_body(x_hbm, src_hbm, dst_hbm,
             agg_hbm, c_hbm, invd_hbm,
             srcB, dstB, widx, rows_st, vals_st, ones_st, zb1,
             deg_sh, invd_sh, c_sh, agg_sh):
    cid = lax.axis_index("c")
    sid = lax.axis_index("s")
    z16 = jnp.zeros((16,), jnp.float32)
    o16 = jnp.ones((16,), jnp.float32)

    def _z1(i, carry):
        zb1[pl.ds(i * 16, 16)] = z16
        return carry
    lax.fori_loop(0, RPT // 16, _z1, 0)
    for j in range(CB // 16):
        ones_st[pl.ds(j * 16, 16)] = o16

    def _zr(i, carry):
        for j in range(D // 16):
            rows_st[i, pl.ds(j * 16, 16)] = z16
        return carry
    lax.fori_loop(0, CB, _zr, 0)

    r0 = sid * RPT
    for k in range(RPT // CB):
        pltpu.sync_copy(rows_st, agg_sh.at[pl.ds(r0 + k * CB, CB)])
    pltpu.sync_copy(zb1, deg_sh.at[pl.ds(r0, RPT)])
    pltpu.sync_copy(zb1, c_sh.at[pl.ds(r0, RPT)])
    plsc.subcore_barrier()

    def _deg(j, carry):
        pltpu.sync_copy(dst_hbm.at[pl.ds((sid * ND + j) * CB, CB)], widx)
        pltpu.sync_copy(ones_st, deg_sh.at[widx], add=True)
        return carry
    lax.fori_loop(0, ND, _deg, 0)
    plsc.subcore_barrier()

    pltpu.sync_copy(deg_sh.at[pl.ds(r0, RPT)], zb1)

    def _inv(i, carry):
        v = zb1[pl.ds(i * 16, 16)]
        zb1[pl.ds(i * 16, 16)] = 1.0 / jnp.maximum(v, 1.0)
        return carry
    lax.fori_loop(0, RPT // 16, _inv, 0)
    pltpu.sync_copy(zb1, invd_sh.at[pl.ds(r0, RPT)])
    plsc.subcore_barrier()

    base = (cid * HALFC + sid * NH) * CB
    pltpu.sync_copy(src_hbm.at[pl.ds(base, NH * CB)], srcB)
    pltpu.sync_copy(dst_hbm.at[pl.ds(base, NH * CB)], dstB)

    def _cp(j, carry):
        pltpu.sync_copy(invd_sh.at[dstB.at[pl.ds(j * CB, CB)]], vals_st)
        pltpu.sync_copy(src_hbm.at[pl.ds(base + j * CB, CB)], widx)
        pltpu.sync_copy(vals_st, c_sh.at[widx], add=True)
        return carry
    lax.fori_loop(0, NH, _cp, 0)

    def _rp(j, carry):
        pltpu.sync_copy(x_hbm.at[srcB.at[pl.ds(j * CB, CB)]], rows_st)
        pltpu.sync_copy(dst_hbm.at[pl.ds(base + j * CB, CB)], widx)
        pltpu.sync_copy(rows_st, agg_sh.at[widx], add=True)
        return carry
    lax.fori_loop(0, NH, _rp, 0)
    plsc.subcore_barrier()

    pltpu.sync_copy(agg_sh.at[pl.ds(r0, RPT)], agg_hbm.at[cid, pl.ds(r0, RPT)])
    pltpu.sync_copy(c_sh.at[pl.ds(r0, RPT)], c_hbm.at[cid, pl.ds(r0, RPT)])

    @pl.when(cid == 0)
    def _():
        pltpu.sync_copy(invd_sh.at[pl.ds(r0, RPT)], invd_hbm.at[pl.ds(r0, RPT)])


_sc_call = pl.kernel(
    _sc_body,
    out_type=(jax.ShapeDtypeStruct((2, NPAD, D), jnp.float32),
              jax.ShapeDtypeStruct((2, NPAD), jnp.float32),
              jax.ShapeDtypeStruct((NPAD,), jnp.float32)),
    mesh=plsc.VectorSubcoreMesh(core_axis_name="c", subcore_axis_name="s"),
    scratch_types=[
        pltpu.VMEM((NH * CB,), jnp.int32),
        pltpu.VMEM((NH * CB,), jnp.int32),
        pltpu.VMEM((CB,), jnp.int32),
        pltpu.VMEM((CB, D), jnp.float32),
        pltpu.VMEM((CB,), jnp.float32),
        pltpu.VMEM((CB,), jnp.float32),
        pltpu.VMEM((RPT,), jnp.float32),
        pltpu.VMEM_SHARED((NPAD,), jnp.float32),
        pltpu.VMEM_SHARED((NPAD,), jnp.float32),
        pltpu.VMEM_SHARED((NPAD,), jnp.float32),
        pltpu.VMEM_SHARED((NPAD, D), jnp.float32),
    ],
)


def _tc_body(x_ref, agg_ref, c_ref, invd_ref, w1r_ref, w1n_ref, b1_ref,
             w2r_ref, w2n_ref, b2_ref, wp_ref, bp_ref,
             out_ref, s0_acc, s1_acc):
    i = pl.program_id(0)

    @pl.when(i == 0)
    def _():
        s0_acc[...] = jnp.zeros_like(s0_acc)
        s1_acc[...] = jnp.zeros_like(s1_acc)

    ri = lax.broadcasted_iota(jnp.int32, (R, R), 0)
    ci = lax.broadcasted_iota(jnp.int32, (R, R), 1)
    dinv = jnp.where(ri == ci, invd_ref[...], 0.0)
    t = jnp.dot(dinv, agg_ref[0] + agg_ref[1],
                preferred_element_type=jnp.float32)
    h = jnp.dot(x_ref[...], w1r_ref[...], preferred_element_type=jnp.float32)
    h += jnp.dot(t, w1n_ref[...], preferred_element_type=jnp.float32)
    h += b1_ref[...]
    h = jnp.maximum(h, 0.0)
    rid = i * R + lax.broadcasted_iota(jnp.int32, (R, 1), 0)
    h = jnp.where(rid < N, h, 0.0)
    cv = c_ref[0] + c_ref[1]
    s0_acc[...] += jnp.sum(h, axis=0, keepdims=True)
    s1_acc[...] += jnp.dot(cv, h, preferred_element_type=jnp.float32)

    @pl.when(i == G - 1)
    def _():
        pooled = jnp.dot(s0_acc[...] * (1.0 / N), w2r_ref[...],
                         preferred_element_type=jnp.float32)
        pooled += jnp.dot(s1_acc[...] * (1.0 / N), w2n_ref[...],
                          preferred_element_type=jnp.float32)
        pooled += b2_ref[...]
        lg = jnp.dot(pooled, wp_ref[...], preferred_element_type=jnp.float32)
        lg += bp_ref[...]
        out_ref[...] = jnp.broadcast_to(lg, (8, 128))


_tc_call = pl.pallas_call(
    _tc_body,
    grid=(G,),
    in_specs=[
        pl.BlockSpec((R, D), lambda i: (i, 0)),
        pl.BlockSpec((2, R, D), lambda i: (0, i, 0)),
        pl.BlockSpec((2, 1, R), lambda i: (0, 0, i)),
        pl.BlockSpec((1, R), lambda i: (0, i)),
        pl.BlockSpec((D, H), lambda i: (0, 0)),
        pl.BlockSpec((D, H), lambda i: (0, 0)),
        pl.BlockSpec((1, H), lambda i: (0, 0)),
        pl.BlockSpec((H, H), lambda i: (0, 0)),
        pl.BlockSpec((H, H), lambda i: (0, 0)),
        pl.BlockSpec((1, H), lambda i: (0, 0)),
        pl.BlockSpec((H, 128), lambda i: (0, 0)),
        pl.BlockSpec((1, 128), lambda i: (0, 0)),
    ],
    out_specs=pl.BlockSpec((8, 128), lambda i: (0, 0)),
    out_shape=jax.ShapeDtypeStruct((8, 128), jnp.float32),
    scratch_shapes=[
        pltpu.VMEM((1, H), jnp.float32),
        pltpu.VMEM((1, H), jnp.float32),
    ],
)


def kernel(x, edge_index, W1r, W1n, b1, W2r, W2n, b2, Wp, bp):
    x_pad = jnp.pad(x, ((0, NPAD - N), (0, 0)))
    pad = N + (jnp.arange(E_PAD - E, dtype=jnp.int32) % (NPAD - N))
    src_p = jnp.concatenate([edge_index[0], pad])
    dst_p = jnp.concatenate([edge_index[1], pad])
    agg, cpart, invd = _sc_call(x_pad, src_p, dst_p)
    c3 = cpart.reshape(2, 1, NPAD)
    wp_pad = jnp.zeros((H, 128), Wp.dtype).at[:, :A].set(Wp)
    bp_pad = jnp.zeros((1, 128), bp.dtype).at[0, :A].set(bp)
    out = _tc_call(x_pad, agg, c3, invd.reshape(1, NPAD), W1r, W1n,
                   b1.reshape(1, H), W2r, W2n, b2.reshape(1, H),
                   wp_pad, bp_pad)
    return out[0, :A]


# speedup vs baseline: 20.2461x; 2.2054x over previous
import jax
import jax.numpy as jnp
from jax import lax
from jax.experimental import pallas as pl
from jax.experimental.pallas import tpu as pltpu
from jax.experimental.pallas import tpu_sc as plsc

N = 10000
E = 320000
D = 128
H = 128
A = 2

CB = 128
NCHUNK = 2560
E_PAD = NCHUNK * CB
NSUB = 16
ND = NCHUNK // NSUB
HALFC = NCHUNK // 2
NH = HALFC // NSUB
NPAD = 10240
RPT = NPAD // NSUB
R = 640
G = NPAD // R


ND_T = NCHUNK // NSUB


def _sc_body(x_hbm, src_hbm, dst_hbm,
             agg_hbm, c_hbm, invd_hbm,
             widx_rs, widx_rd, widx_d, widx_cd, widx_cs, rows_st, vals_st,
             ones_st, zb1,
             deg_sh, invd_sh, c_sh, agg_sh,
             sem_ws, sem_rd, sem_gr, sem_sr, sem_dw, sem_ds,
             sem_cd, sem_cs, sem_sc):
    cid = lax.axis_index("c")
    sid = lax.axis_index("s")
    z16 = jnp.zeros((16,), jnp.float32)
    o16 = jnp.ones((16,), jnp.float32)

    def _z1(i, carry):
        zb1[pl.ds(i * 16, 16)] = z16
        return carry
    lax.fori_loop(0, RPT // 16, _z1, 0)
    for j in range(CB // 16):
        ones_st[pl.ds(j * 16, 16)] = o16

    def _zr(i, carry):
        for j in range(D // 16):
            rows_st[0, i, pl.ds(j * 16, 16)] = z16
        return carry
    lax.fori_loop(0, CB, _zr, 0)

    r0 = sid * RPT
    for k in range(RPT // CB):
        pltpu.sync_copy(rows_st.at[0], agg_sh.at[pl.ds(r0 + k * CB, CB)])
    pltpu.sync_copy(zb1, deg_sh.at[pl.ds(r0, RPT)])
    pltpu.sync_copy(zb1, c_sh.at[pl.ds(r0, RPT)])
    plsc.subcore_barrier()

    base = (cid * HALFC + sid * NH) * CB
    dbase = sid * ND_T * CB

    def _wait(desc_src, dst, sem):
        pltpu.make_async_copy(desc_src, dst, sem).wait()

    def _fire_sfetch(j, s):
        pltpu.async_copy(src_hbm.at[pl.ds(base + j * CB, CB)],
                         widx_rs.at[s], sem_ws.at[s])

    def _fire_dfetch(j, s):
        pltpu.async_copy(dst_hbm.at[pl.ds(base + j * CB, CB)],
                         widx_rd.at[s], sem_rd.at[s])

    def _fire_degf(dg, s):
        pltpu.async_copy(dst_hbm.at[pl.ds(dbase + dg * CB, CB)],
                         widx_d.at[s], sem_dw.at[s])

    for s in range(4):
        _fire_sfetch(s, s)
        _fire_dfetch(s, s)
        _fire_degf(s, s)
    for b in range(2):
        _wait(src_hbm.at[pl.ds(0, CB)], widx_rs.at[b], sem_ws.at[b])
        pltpu.async_copy(x_hbm.at[widx_rs.at[b]], rows_st.at[b],
                         sem_gr.at[b])

    def _phase_a(oo, carry):
        for t in range(8):
            j = oo * 8 + t
            b = t % 2
            s = t % 4
            s2 = (t + 2) % 4
            _wait(x_hbm.at[widx_rs.at[s]], rows_st.at[b], sem_gr.at[b])
            _wait(dst_hbm.at[pl.ds(0, CB)], widx_rd.at[s], sem_rd.at[s])
            pltpu.async_copy(rows_st.at[b], agg_sh.at[widx_rd.at[s]],
                             sem_sr.at[b], add=True)
            _wait(rows_st.at[b], agg_sh.at[widx_rd.at[s]], sem_sr.at[b])

            @pl.when(j + 4 < NH)
            def _():
                _fire_sfetch(j + 4, s)
                _fire_dfetch(j + 4, s)

            @pl.when(j + 2 < NH)
            def _():
                _wait(src_hbm.at[pl.ds(0, CB)], widx_rs.at[s2], sem_ws.at[s2])
                pltpu.async_copy(x_hbm.at[widx_rs.at[s2]],
                                 rows_st.at[b], sem_gr.at[b])
            for u in range(2):
                dg = j * 2 + u
                db = (t * 2 + u) % 4
                db2 = (t * 2 + u + 2) % 4
                _wait(dst_hbm.at[pl.ds(0, CB)], widx_d.at[db], sem_dw.at[db])
                pltpu.async_copy(ones_st, deg_sh.at[widx_d.at[db]],
                                 sem_ds.at[db], add=True)

                @pl.when(dg >= 2)
                def _():
                    _wait(ones_st, deg_sh.at[widx_d.at[db2]], sem_ds.at[db2])

                @pl.when((dg + 2 >= 4) & (dg + 2 < ND_T))
                def _():
                    _fire_degf(dg + 2, db2)
        return carry
    lax.fori_loop(0, NH // 8, _phase_a, 0)
    for u in range(2):
        db = (ND_T - 2 + u) % 4
        _wait(ones_st, deg_sh.at[widx_d.at[db]], sem_ds.at[db])
    plsc.subcore_barrier()

    pltpu.sync_copy(deg_sh.at[pl.ds(r0, RPT)], zb1)

    def _inv(i, carry):
        v = zb1[pl.ds(i * 16, 16)]
        zb1[pl.ds(i * 16, 16)] = 1.0 / jnp.maximum(v, 1.0)
        return carry
    lax.fori_loop(0, RPT // 16, _inv, 0)
    pltpu.sync_copy(zb1, invd_sh.at[pl.ds(r0, RPT)])
    plsc.subcore_barrier()

    def _fire_cdf(g, s):
        pltpu.async_copy(dst_hbm.at[pl.ds(base + g * CB, CB)],
                         widx_cd.at[s], sem_cd.at[s])

    def _fire_csf(g, s):
        pltpu.async_copy(src_hbm.at[pl.ds(base + g * CB, CB)],
                         widx_cs.at[s], sem_cs.at[s])

    for s in range(2):
        _fire_cdf(s, s)
        _fire_csf(s, s)

    def _phase_b(gg, carry):
        for u in range(4):
            g = gg * 4 + u
            vb = u % 2
            scd = u % 2
            scs = u
            scs2 = (u + 2) % 4

            @pl.when(g >= 2)
            def _():
                _wait(vals_st.at[vb], c_sh.at[widx_cs.at[scs2]],
                      sem_sc.at[vb])

            @pl.when(g + 2 < NH)
            def _():
                _fire_csf(g + 2, scs2)
            _wait(dst_hbm.at[pl.ds(0, CB)], widx_cd.at[scd], sem_cd.at[scd])
            pltpu.async_copy(invd_sh.at[widx_cd.at[scd]], vals_st.at[vb],
                             sem_gr.at[vb])
            _wait(invd_sh.at[widx_cd.at[scd]], vals_st.at[vb], sem_gr.at[vb])
            _wait(src_hbm.at[pl.ds(0, CB)], widx_cs.at[scs], sem_cs.at[scs])
            pltpu.async_copy(vals_st.at[vb], c_sh.at[widx_cs.at[scs]],
                             sem_sc.at[vb], add=True)

            @pl.when(g + 2 < NH)
            def _():
                _fire_cdf(g + 2, scd)
        return carry
    lax.fori_loop(0, NH // 4, _phase_b, 0)
    for u in range(2):
        g = NH - 2 + u
        _wait(vals_st.at[g % 2], c_sh.at[widx_cs.at[g % 4]], sem_sc.at[g % 2])
    plsc.subcore_barrier()

    pltpu.sync_copy(agg_sh.at[pl.ds(r0, RPT)], agg_hbm.at[cid, pl.ds(r0, RPT)])
    pltpu.sync_copy(c_sh.at[pl.ds(r0, RPT)], c_hbm.at[cid, pl.ds(r0, RPT)])

    @pl.when(cid == 0)
    def _():
        pltpu.sync_copy(invd_sh.at[pl.ds(r0, RPT)], invd_hbm.at[pl.ds(r0, RPT)])


_sc_call = pl.kernel(
    _sc_body,
    out_type=(jax.ShapeDtypeStruct((2, NPAD, D), jnp.float32),
              jax.ShapeDtypeStruct((2, NPAD), jnp.float32),
              jax.ShapeDtypeStruct((NPAD,), jnp.float32)),
    mesh=plsc.VectorSubcoreMesh(core_axis_name="c", subcore_axis_name="s"),
    scratch_types=[
        pltpu.VMEM((4, CB), jnp.int32),
        pltpu.VMEM((4, CB), jnp.int32),
        pltpu.VMEM((4, CB), jnp.int32),
        pltpu.VMEM((4, CB), jnp.int32),
        pltpu.VMEM((4, CB), jnp.int32),
        pltpu.VMEM((2, CB, D), jnp.float32),
        pltpu.VMEM((2, CB), jnp.float32),
        pltpu.VMEM((CB,), jnp.float32),
        pltpu.VMEM((RPT,), jnp.float32),
        pltpu.VMEM_SHARED((NPAD,), jnp.float32),
        pltpu.VMEM_SHARED((NPAD,), jnp.float32),
        pltpu.VMEM_SHARED((NPAD,), jnp.float32),
        pltpu.VMEM_SHARED((NPAD, D), jnp.float32),
        pltpu.SemaphoreType.DMA((4,)),
        pltpu.SemaphoreType.DMA((4,)),
        pltpu.SemaphoreType.DMA((2,)),
        pltpu.SemaphoreType.DMA((2,)),
        pltpu.SemaphoreType.DMA((4,)),
        pltpu.SemaphoreType.DMA((4,)),
        pltpu.SemaphoreType.DMA((2,)),
        pltpu.SemaphoreType.DMA((4,)),
        pltpu.SemaphoreType.DMA((2,)),
    ],)


def _tc_body(x_ref, agg_ref, c_ref, invd_ref, w1r_ref, w1n_ref, b1_ref,
             w2r_ref, w2n_ref, b2_ref, wp_ref, bp_ref,
             out_ref, s0_acc, s1_acc):
    i = pl.program_id(0)

    @pl.when(i == 0)
    def _():
        s0_acc[...] = jnp.zeros_like(s0_acc)
        s1_acc[...] = jnp.zeros_like(s1_acc)

    ri = lax.broadcasted_iota(jnp.int32, (R, R), 0)
    ci = lax.broadcasted_iota(jnp.int32, (R, R), 1)
    dinv = jnp.where(ri == ci, invd_ref[...], 0.0)
    t = jnp.dot(dinv, agg_ref[0] + agg_ref[1],
                preferred_element_type=jnp.float32)
    h = jnp.dot(x_ref[...], w1r_ref[...], preferred_element_type=jnp.float32)
    h += jnp.dot(t, w1n_ref[...], preferred_element_type=jnp.float32)
    h += b1_ref[...]
    h = jnp.maximum(h, 0.0)
    rid = i * R + lax.broadcasted_iota(jnp.int32, (R, 1), 0)
    h = jnp.where(rid < N, h, 0.0)
    cv = c_ref[0] + c_ref[1]
    s0_acc[...] += jnp.sum(h, axis=0, keepdims=True)
    s1_acc[...] += jnp.dot(cv, h, preferred_element_type=jnp.float32)

    @pl.when(i == G - 1)
    def _():
        pooled = jnp.dot(s0_acc[...] * (1.0 / N), w2r_ref[...],
                         preferred_element_type=jnp.float32)
        pooled += jnp.dot(s1_acc[...] * (1.0 / N), w2n_ref[...],
                          preferred_element_type=jnp.float32)
        pooled += b2_ref[...]
        lg = jnp.dot(pooled, wp_ref[...], preferred_element_type=jnp.float32)
        lg += bp_ref[...]
        out_ref[...] = jnp.broadcast_to(lg, (8, 128))


_tc_call = pl.pallas_call(
    _tc_body,
    grid=(G,),
    in_specs=[
        pl.BlockSpec((R, D), lambda i: (i, 0)),
        pl.BlockSpec((2, R, D), lambda i: (0, i, 0)),
        pl.BlockSpec((2, 1, R), lambda i: (0, 0, i)),
        pl.BlockSpec((1, R), lambda i: (0, i)),
        pl.BlockSpec((D, H), lambda i: (0, 0)),
        pl.BlockSpec((D, H), lambda i: (0, 0)),
        pl.BlockSpec((1, H), lambda i: (0, 0)),
        pl.BlockSpec((H, H), lambda i: (0, 0)),
        pl.BlockSpec((H, H), lambda i: (0, 0)),
        pl.BlockSpec((1, H), lambda i: (0, 0)),
        pl.BlockSpec((H, 128), lambda i: (0, 0)),
        pl.BlockSpec((1, 128), lambda i: (0, 0)),
    ],
    out_specs=pl.BlockSpec((8, 128), lambda i: (0, 0)),
    out_shape=jax.ShapeDtypeStruct((8, 128), jnp.float32),
    scratch_shapes=[
        pltpu.VMEM((1, H), jnp.float32),
        pltpu.VMEM((1, H), jnp.float32),
    ],
)


def kernel(x, edge_index, W1r, W1n, b1, W2r, W2n, b2, Wp, bp):
    x_pad = jnp.pad(x, ((0, NPAD - N), (0, 0)))
    pad = N + (jnp.arange(E_PAD - E, dtype=jnp.int32) % (NPAD - N))
    src_p = jnp.concatenate([edge_index[0], pad])
    dst_p = jnp.concatenate([edge_index[1], pad])
    agg, cpart, invd = _sc_call(x_pad, src_p, dst_p)
    c3 = cpart.reshape(2, 1, NPAD)
    wp_pad = jnp.zeros((H, 128), Wp.dtype).at[:, :A].set(Wp)
    bp_pad = jnp.zeros((1, 128), bp.dtype).at[0, :A].set(bp)
    out = _tc_call(x_pad, agg, c3, invd.reshape(1, NPAD), W1r, W1n,
                   b1.reshape(1, H), W2r, W2n, b2.reshape(1, H),
                   wp_pad, bp_pad)
    return out[0, :A]


# speedup vs baseline: 20.5415x; 1.0146x over previous
import jax
import jax.numpy as jnp
from jax import lax
from jax.experimental import pallas as pl
from jax.experimental.pallas import tpu as pltpu
from jax.experimental.pallas import tpu_sc as plsc

N = 10000
E = 320000
D = 128
H = 128
A = 2

CB = 128
NCHUNK = 2560
E_PAD = NCHUNK * CB
NSUB = 16
ND = NCHUNK // NSUB
HALFC = NCHUNK // 2
NH = HALFC // NSUB
NPAD = 10240
RPT = NPAD // NSUB
R = 640
G = NPAD // R


ND_T = NCHUNK // NSUB


def _sc_body(x_hbm, src_hbm, dst_hbm,
             agg_hbm, c_hbm, invd_hbm,
             widx_rs, widx_rd, widx_d, widx_cd, widx_cs, rows_st, vals_st,
             ones_st, zb1,
             deg_sh, invd_sh, c_sh, agg_sh,
             sem_ws, sem_rd, sem_gr, sem_sr, sem_dw, sem_ds,
             sem_cd, sem_cs, sem_sc):
    cid = lax.axis_index("c")
    sid = lax.axis_index("s")
    z16 = jnp.zeros((16,), jnp.float32)
    o16 = jnp.ones((16,), jnp.float32)

    def _z1(i, carry):
        zb1[pl.ds(i * 16, 16)] = z16
        return carry
    lax.fori_loop(0, RPT // 16, _z1, 0)
    for j in range(CB // 16):
        ones_st[pl.ds(j * 16, 16)] = o16

    def _zr(i, carry):
        for j in range(D // 16):
            rows_st[0, i, pl.ds(j * 16, 16)] = z16
        return carry
    lax.fori_loop(0, CB, _zr, 0)

    r0 = sid * RPT
    for k in range(RPT // CB):
        pltpu.sync_copy(rows_st.at[0], agg_sh.at[pl.ds(r0 + k * CB, CB)])
    pltpu.sync_copy(zb1, deg_sh.at[pl.ds(r0, RPT)])
    pltpu.sync_copy(zb1, c_sh.at[pl.ds(r0, RPT)])
    plsc.subcore_barrier()

    base = (cid * HALFC + sid * NH) * CB
    dbase = sid * ND_T * CB

    def _wait(desc_src, dst, sem):
        pltpu.make_async_copy(desc_src, dst, sem).wait()

    def _fire_sfetch(j, s):
        pltpu.async_copy(src_hbm.at[pl.ds(base + j * CB, CB)],
                         widx_rs.at[s], sem_ws.at[s])

    def _fire_dfetch(j, s):
        pltpu.async_copy(dst_hbm.at[pl.ds(base + j * CB, CB)],
                         widx_rd.at[s], sem_rd.at[s])

    def _fire_degf(dg, s):
        pltpu.async_copy(dst_hbm.at[pl.ds(dbase + dg * CB, CB)],
                         widx_d.at[s], sem_dw.at[s])

    for s in range(4):
        _fire_sfetch(s, s)
        _fire_dfetch(s, s)
        _fire_degf(s, s)
    for b in range(2):
        _wait(src_hbm.at[pl.ds(0, CB)], widx_rs.at[b], sem_ws.at[b])
        pltpu.async_copy(x_hbm.at[widx_rs.at[b]], rows_st.at[b],
                         sem_gr.at[b])

    def _phase_a(oo, carry):
        for t in range(8):
            j = oo * 8 + t
            b = t % 2
            s = t % 4
            s2 = (t + 2) % 4
            _wait(x_hbm.at[widx_rs.at[s]], rows_st.at[b], sem_gr.at[b])
            _wait(dst_hbm.at[pl.ds(0, CB)], widx_rd.at[s], sem_rd.at[s])
            pltpu.async_copy(rows_st.at[b], agg_sh.at[widx_rd.at[s]],
                             sem_sr.at[b], add=True)
            _wait(rows_st.at[b], agg_sh.at[widx_rd.at[s]], sem_sr.at[b])

            @pl.when(j + 4 < NH)
            def _():
                _fire_sfetch(j + 4, s)
                _fire_dfetch(j + 4, s)

            @pl.when(j + 2 < NH)
            def _():
                _wait(src_hbm.at[pl.ds(0, CB)], widx_rs.at[s2], sem_ws.at[s2])
                pltpu.async_copy(x_hbm.at[widx_rs.at[s2]],
                                 rows_st.at[b], sem_gr.at[b])
            for u in range(2):
                dg = j * 2 + u
                db = (t * 2 + u) % 4
                db2 = (t * 2 + u + 2) % 4
                _wait(dst_hbm.at[pl.ds(0, CB)], widx_d.at[db], sem_dw.at[db])
                pltpu.async_copy(ones_st, deg_sh.at[widx_d.at[db]],
                                 sem_ds.at[db], add=True)

                @pl.when(dg >= 2)
                def _():
                    _wait(ones_st, deg_sh.at[widx_d.at[db2]], sem_ds.at[db2])

                @pl.when((dg + 2 >= 4) & (dg + 2 < ND_T))
                def _():
                    _fire_degf(dg + 2, db2)
        return carry
    lax.fori_loop(0, NH // 8, _phase_a, 0)
    for u in range(2):
        db = (ND_T - 2 + u) % 4
        _wait(ones_st, deg_sh.at[widx_d.at[db]], sem_ds.at[db])
    plsc.subcore_barrier()

    pltpu.sync_copy(deg_sh.at[pl.ds(r0, RPT)], zb1)

    def _inv(i, carry):
        v = zb1[pl.ds(i * 16, 16)]
        zb1[pl.ds(i * 16, 16)] = 1.0 / jnp.maximum(v, 1.0)
        return carry
    lax.fori_loop(0, RPT // 16, _inv, 0)
    pltpu.sync_copy(zb1, invd_sh.at[pl.ds(r0, RPT)])
    plsc.subcore_barrier()

    def _fire_cdf(g, s):
        pltpu.async_copy(dst_hbm.at[pl.ds(base + g * CB, CB)],
                         widx_cd.at[s], sem_cd.at[s])

    def _fire_csf(g, s):
        pltpu.async_copy(src_hbm.at[pl.ds(base + g * CB, CB)],
                         widx_cs.at[s], sem_cs.at[s])

    for s in range(2):
        _fire_cdf(s, s)
        _fire_csf(s, s)

    def _phase_b(gg, carry):
        for u in range(4):
            g = gg * 4 + u
            vb = u % 2
            scd = u % 2
            scs = u
            scs2 = (u + 2) % 4

            @pl.when(g >= 2)
            def _():
                _wait(vals_st.at[vb], c_sh.at[widx_cs.at[scs2]],
                      sem_sc.at[vb])

            @pl.when(g + 2 < NH)
            def _():
                _fire_csf(g + 2, scs2)
            _wait(dst_hbm.at[pl.ds(0, CB)], widx_cd.at[scd], sem_cd.at[scd])
            pltpu.async_copy(invd_sh.at[widx_cd.at[scd]], vals_st.at[vb],
                             sem_gr.at[vb])
            _wait(invd_sh.at[widx_cd.at[scd]], vals_st.at[vb], sem_gr.at[vb])
            _wait(src_hbm.at[pl.ds(0, CB)], widx_cs.at[scs], sem_cs.at[scs])
            pltpu.async_copy(vals_st.at[vb], c_sh.at[widx_cs.at[scs]],
                             sem_sc.at[vb], add=True)

            @pl.when(g + 2 < NH)
            def _():
                _fire_cdf(g + 2, scd)
        return carry
    lax.fori_loop(0, NH // 4, _phase_b, 0)
    for u in range(2):
        g = NH - 2 + u
        _wait(vals_st.at[g % 2], c_sh.at[widx_cs.at[g % 4]], sem_sc.at[g % 2])
    plsc.subcore_barrier()

    pltpu.sync_copy(agg_sh.at[pl.ds(r0, RPT)], agg_hbm.at[cid, pl.ds(r0, RPT)])
    pltpu.sync_copy(c_sh.at[pl.ds(r0, RPT)], c_hbm.at[cid, pl.ds(r0, RPT)])

    @pl.when(cid == 0)
    def _():
        pltpu.sync_copy(invd_sh.at[pl.ds(r0, RPT)], invd_hbm.at[pl.ds(r0, RPT)])


_sc_call = pl.kernel(
    _sc_body,
    out_type=(jax.ShapeDtypeStruct((2, NPAD, D), jnp.float32),
              jax.ShapeDtypeStruct((2, NPAD), jnp.float32),
              jax.ShapeDtypeStruct((NPAD,), jnp.float32)),
    mesh=plsc.VectorSubcoreMesh(core_axis_name="c", subcore_axis_name="s"),
    scratch_types=[
        pltpu.VMEM((4, CB), jnp.int32),
        pltpu.VMEM((4, CB), jnp.int32),
        pltpu.VMEM((4, CB), jnp.int32),
        pltpu.VMEM((4, CB), jnp.int32),
        pltpu.VMEM((4, CB), jnp.int32),
        pltpu.VMEM((2, CB, D), jnp.float32),
        pltpu.VMEM((2, CB), jnp.float32),
        pltpu.VMEM((CB,), jnp.float32),
        pltpu.VMEM((RPT,), jnp.float32),
        pltpu.VMEM_SHARED((NPAD,), jnp.float32),
        pltpu.VMEM_SHARED((NPAD,), jnp.float32),
        pltpu.VMEM_SHARED((NPAD,), jnp.float32),
        pltpu.VMEM_SHARED((NPAD, D), jnp.float32),
        pltpu.SemaphoreType.DMA((4,)),
        pltpu.SemaphoreType.DMA((4,)),
        pltpu.SemaphoreType.DMA((2,)),
        pltpu.SemaphoreType.DMA((2,)),
        pltpu.SemaphoreType.DMA((4,)),
        pltpu.SemaphoreType.DMA((4,)),
        pltpu.SemaphoreType.DMA((2,)),
        pltpu.SemaphoreType.DMA((4,)),
        pltpu.SemaphoreType.DMA((2,)),
    ],)


def _tc_body(x_ref, agg_ref, c_ref, invd_ref, w1r_ref, w1n_ref, b1_ref,
             w2r_ref, w2n_ref, b2_ref, wp_ref, bp_ref,
             out_ref, s0_acc, s1_acc):
    i = pl.program_id(0)

    @pl.when(i == 0)
    def _():
        s0_acc[...] = jnp.zeros_like(s0_acc)
        s1_acc[...] = jnp.zeros_like(s1_acc)

    ri = lax.broadcasted_iota(jnp.int32, (R, R), 0)
    ci = lax.broadcasted_iota(jnp.int32, (R, R), 1)
    dinv = jnp.where(ri == ci, invd_ref[...], 0.0)
    t = jnp.dot(dinv, agg_ref[0] + agg_ref[1],
                preferred_element_type=jnp.float32)
    h = jnp.dot(x_ref[...], w1r_ref[...], preferred_element_type=jnp.float32)
    h += jnp.dot(t, w1n_ref[...], preferred_element_type=jnp.float32)
    h += b1_ref[...]
    h = jnp.maximum(h, 0.0)
    rid = i * R + lax.broadcasted_iota(jnp.int32, (R, 1), 0)
    h = jnp.where(rid < N, h, 0.0)
    cv = c_ref[0] + c_ref[1]
    s0_acc[...] += jnp.sum(h, axis=0, keepdims=True)
    s1_acc[...] += jnp.dot(cv, h, preferred_element_type=jnp.float32)

    @pl.when(i == G - 1)
    def _():
        pooled = jnp.dot(s0_acc[...] * (1.0 / N), w2r_ref[...],
                         preferred_element_type=jnp.float32)
        pooled += jnp.dot(s1_acc[...] * (1.0 / N), w2n_ref[...],
                          preferred_element_type=jnp.float32)
        pooled += b2_ref[...]
        lg = jnp.dot(pooled, wp_ref[...], preferred_element_type=jnp.float32)
        lg += bp_ref[...]
        out_ref[...] = jnp.broadcast_to(lg, (8, 128))


_tc_call = pl.pallas_call(
    _tc_body,
    grid=(G,),
    in_specs=[
        pl.BlockSpec((R, D), lambda i: (i, 0)),
        pl.BlockSpec((2, R, D), lambda i: (0, i, 0)),
        pl.BlockSpec((2, 1, R), lambda i: (0, 0, i)),
        pl.BlockSpec((1, R), lambda i: (0, i)),
        pl.BlockSpec((D, H), lambda i: (0, 0)),
        pl.BlockSpec((D, H), lambda i: (0, 0)),
        pl.BlockSpec((1, H), lambda i: (0, 0)),
        pl.BlockSpec((H, H), lambda i: (0, 0)),
        pl.BlockSpec((H, H), lambda i: (0, 0)),
        pl.BlockSpec((1, H), lambda i: (0, 0)),
        pl.BlockSpec((H, 128), lambda i: (0, 0)),
        pl.BlockSpec((1, 128), lambda i: (0, 0)),
    ],
    out_specs=pl.BlockSpec((8, 128), lambda i: (0, 0)),
    out_shape=jax.ShapeDtypeStruct((8, 128), jnp.float32),
    scratch_shapes=[
        pltpu.VMEM((1, H), jnp.float32),
        pltpu.VMEM((1, H), jnp.float32),
    ],
)


def kernel(x, edge_index, W1r, W1n, b1, W2r, W2n, b2, Wp, bp):
    npd = E_PAD - E
    pad_s = jnp.arange(npd, dtype=jnp.int32) % N
    pad_d = N + (jnp.arange(npd, dtype=jnp.int32) % (NPAD - N))
    src_p = jnp.concatenate([edge_index[0], pad_s])
    dst_p = jnp.concatenate([edge_index[1], pad_d])
    agg, cpart, invd = _sc_call(x, src_p, dst_p)
    c3 = cpart.reshape(2, 1, NPAD)
    wp_pad = jnp.zeros((H, 128), Wp.dtype).at[:, :A].set(Wp)
    bp_pad = jnp.zeros((1, 128), bp.dtype).at[0, :A].set(bp)
    out = _tc_call(x, agg, c3, invd.reshape(1, NPAD), W1r, W1n,
                   b1.reshape(1, H), W2r, W2n, b2.reshape(1, H),
                   wp_pad, bp_pad)
    return out[0, :A]
